# baseline (speedup 1.0000x reference)
import jax
import jax.numpy as jnp
from jax import lax
from jax.experimental import pallas as pl
from jax.experimental.pallas import tpu as pltpu
from jax.experimental.pallas import tpu_sc as plsc

_N = 10000
_NP = 10240
_E = 160000
_D1 = 256
_D2 = 128

_NC = 2
_NS = 16
_NW = _NC * _NS
_CH = 128
_EP = 163840
_NCH = _EP // _CH
_CPT = _NCH // _NW
_HPT = _CPT // 2
_RPT = _NP // _NS

_RB3 = 10
_RB = _RB3 * 128
_GRID = _NP // _RB

_mesh = plsc.VectorSubcoreMesh(core_axis_name="c", subcore_axis_name="s")


def _deg_body(src_hbm, dst_hbm, hout_hbm, hin_hbm, idxs_v, idxd_v, ho_v, hi_v):
    c = lax.axis_index("c")
    s = lax.axis_index("s")
    wid = c * _NS + s
    z16 = jnp.zeros((16,), jnp.float32)

    def zero(i, _):
        ho_v[pl.ds(i * 16, 16)] = z16
        hi_v[pl.ds(i * 16, 16)] = z16
        return 0

    lax.fori_loop(0, _NP // 16, zero, 0)

    pltpu.sync_copy(src_hbm.at[pl.ds(wid * _CPT, _CPT)], idxs_v)
    pltpu.sync_copy(dst_hbm.at[pl.ds(wid * _CPT, _CPT)], idxd_v)

    ones16 = jnp.ones((16,), jnp.float32)

    def body(t, _):
        for j in range(_CH // 16):
            plsc.addupdate_scatter(ho_v, [idxs_v[t, pl.ds(j * 16, 16)]], ones16)
            plsc.addupdate_scatter(hi_v, [idxd_v[t, pl.ds(j * 16, 16)]], ones16)
        return 0

    lax.fori_loop(0, _CPT, body, 0)
    pltpu.sync_copy(ho_v, hout_hbm.at[wid])
    pltpu.sync_copy(hi_v, hin_hbm.at[wid])


_deg = pl.kernel(
    _deg_body,
    out_type=(
        jax.ShapeDtypeStruct((_NW, _NP), jnp.float32),
        jax.ShapeDtypeStruct((_NW, _NP), jnp.float32),
    ),
    mesh=_mesh,
    compiler_params=pltpu.CompilerParams(needs_layout_passes=False),
    scratch_types=[
        pltpu.VMEM((_CPT, _CH), jnp.int32),
        pltpu.VMEM((_CPT, _CH), jnp.int32),
        pltpu.VMEM((_NP,), jnp.float32),
        pltpu.VMEM((_NP,), jnp.float32),
    ],
)


def _agg_body(x_hbm, src_hbm, dst_hbm, out_hbm,
              idxs_v, idxd_v, buf0, buf1, acc_sh, gs0, gs1, ss0, ss1):
    c = lax.axis_index("c")
    s = lax.axis_index("s")
    wid = c * _NS + s
    cb = wid * _CPT
    z16 = jnp.zeros((16,), jnp.float32)

    def zero(i, _):
        for j in range(_D2 // 16):
            buf0[i, pl.ds(j * 16, 16)] = z16
        return 0

    lax.fori_loop(0, _CH, zero, 0)

    pltpu.sync_copy(src_hbm.at[pl.ds(cb, _CPT)], idxs_v)
    pltpu.sync_copy(dst_hbm.at[pl.ds(cb, _CPT)], idxd_v)

    base = s * _RPT
    for k in range(_RPT // _CH):
        pltpu.sync_copy(buf0, acc_sh.at[pl.ds(base + k * _CH, _CH)])
    plsc.subcore_barrier()

    pltpu.async_copy(x_hbm.at[idxs_v.at[0]], buf0, gs0)

    def body(t, _):
        j0 = 2 * t
        j1 = 2 * t + 1
        pltpu.make_async_copy(x_hbm.at[idxs_v.at[j0]], buf0, gs0).wait()
        h_s0 = pltpu.async_copy(buf0, acc_sh.at[idxd_v.at[j0]], ss0, add=True)
        h_g1 = pltpu.async_copy(x_hbm.at[idxs_v.at[j1]], buf1, gs1)
        h_g1.wait()
        h_s1 = pltpu.async_copy(buf1, acc_sh.at[idxd_v.at[j1]], ss1, add=True)
        h_s0.wait()

        @pl.when(t + 1 < _HPT)
        def _():
            pltpu.async_copy(x_hbm.at[idxs_v.at[j0 + 2]], buf0, gs0)

        h_s1.wait()
        return 0

    lax.fori_loop(0, _HPT, body, 0)
    plsc.subcore_barrier()
    pltpu.sync_copy(acc_sh.at[pl.ds(base, _RPT)], out_hbm.at[c, pl.ds(base, _RPT)])


_agg = pl.kernel(
    _agg_body,
    out_type=jax.ShapeDtypeStruct((_NC, _NP, _D2), jnp.float32),
    mesh=_mesh,
    scratch_types=[
        pltpu.VMEM((_CPT, _CH), jnp.int32),
        pltpu.VMEM((_CPT, _CH), jnp.int32),
        pltpu.VMEM((_CH, _D2), jnp.float32),
        pltpu.VMEM((_CH, _D2), jnp.float32),
        pltpu.VMEM_SHARED((_NP, _D2), jnp.float32),
        pltpu.SemaphoreType.DMA,
        pltpu.SemaphoreType.DMA,
        pltpu.SemaphoreType.DMA,
        pltpu.SemaphoreType.DMA,
    ],
)


def _mm1_body(hout_ref, hin_ref, x_ref, w_ref, y_ref, so_ref, si_ref):
    do = jnp.sum(hout_ref[...], axis=1)
    di = jnp.sum(hin_ref[...], axis=1)
    so = lax.rsqrt(jnp.maximum(do, 1.0))
    si = lax.rsqrt(jnp.maximum(di, 1.0))
    so_ref[...] = so[None]
    si_ref[...] = si[None]
    x = x_ref[...] * so[:, :, None]
    y_ref[...] = jnp.dot(
        x.reshape(_RB, _D1), w_ref[...], preferred_element_type=jnp.float32
    )


def _mm1(hout3, hin3, feats3, w1):
    return pl.pallas_call(
        _mm1_body,
        grid=(_GRID,),
        in_specs=[
            pl.BlockSpec((_RB3, _NW, 128), lambda b: (b, 0, 0)),
            pl.BlockSpec((_RB3, _NW, 128), lambda b: (b, 0, 0)),
            pl.BlockSpec((_RB3, 128, _D1), lambda b: (b, 0, 0)),
            pl.BlockSpec((_D1, _D2), lambda b: (0, 0)),
        ],
        out_specs=[
            pl.BlockSpec((_RB, _D2), lambda b: (b, 0)),
            pl.BlockSpec((1, _RB3, 128), lambda b: (b, 0, 0)),
            pl.BlockSpec((1, _RB3, 128), lambda b: (b, 0, 0)),
        ],
        out_shape=[
            jax.ShapeDtypeStruct((_NP, _D2), jnp.float32),
            jax.ShapeDtypeStruct((_GRID, _RB3, 128), jnp.float32),
            jax.ShapeDtypeStruct((_GRID, _RB3, 128), jnp.float32),
        ],
    )(hout3, hin3, feats3, w1)


def _mm2_body(p0_ref, p1_ref, si_ref, so_ref, b1_ref, w_ref, y_ref):
    p = (p0_ref[...] + p1_ref[...]).reshape(_RB3, 128, _D2)
    si = si_ref[0]
    so = so_ref[0]
    b = b1_ref[...]
    h = jnp.maximum(p * si[:, :, None] + b[0][None, None, :], 0.0)
    h = h * so[:, :, None]
    y_ref[...] = jnp.dot(
        h.reshape(_RB, _D2), w_ref[...], preferred_element_type=jnp.float32
    )


def _mm2(p0, p1, si, so, b1_2d, w2):
    return pl.pallas_call(
        _mm2_body,
        grid=(_GRID,),
        in_specs=[
            pl.BlockSpec((_RB, _D2), lambda b: (b, 0)),
            pl.BlockSpec((_RB, _D2), lambda b: (b, 0)),
            pl.BlockSpec((1, _RB3, 128), lambda b: (b, 0, 0)),
            pl.BlockSpec((1, _RB3, 128), lambda b: (b, 0, 0)),
            pl.BlockSpec((1, _D2), lambda b: (0, 0)),
            pl.BlockSpec((_D2, _D2), lambda b: (0, 0)),
        ],
        out_specs=pl.BlockSpec((_RB, _D2), lambda b: (b, 0)),
        out_shape=jax.ShapeDtypeStruct((_NP, _D2), jnp.float32),
    )(p0, p1, si, so, b1_2d, w2)


def _out_body(q0_ref, q1_ref, si_ref, b2_ref, o_ref):
    q = (q0_ref[...] + q1_ref[...]).reshape(_RB3, 128, _D2)
    si = si_ref[0]
    b = b2_ref[...]
    o = jnp.maximum(q * si[:, :, None] + b[0][None, None, :], 0.0)
    o_ref[...] = o.reshape(_RB, _D2)


def _out(q0, q1, si, b2_2d):
    return pl.pallas_call(
        _out_body,
        grid=(_GRID,),
        in_specs=[
            pl.BlockSpec((_RB, _D2), lambda b: (b, 0)),
            pl.BlockSpec((_RB, _D2), lambda b: (b, 0)),
            pl.BlockSpec((1, _RB3, 128), lambda b: (b, 0, 0)),
            pl.BlockSpec((1, _D2), lambda b: (0, 0)),
        ],
        out_specs=pl.BlockSpec((_RB, _D2), lambda b: (b, 0)),
        out_shape=jax.ShapeDtypeStruct((_NP, _D2), jnp.float32),
    )(q0, q1, si, b2_2d)


def kernel(feats, edge_index, W1, b1, W2, b2):
    src = edge_index[0].astype(jnp.int32)
    dst = edge_index[1].astype(jnp.int32)
    pad = jnp.full((_EP - _E,), _N, jnp.int32)
    src2 = jnp.concatenate([src, pad]).reshape(_NCH, _CH)
    dst2 = jnp.concatenate([dst, pad]).reshape(_NCH, _CH)
    feats3 = jnp.pad(feats, ((0, _NP - _N), (0, 0))).reshape(_NP // 128, 128, _D1)

    hout, hin = _deg(src2, dst2)
    hout3 = hout.reshape(_NW, _NP // 128, 128).transpose(1, 0, 2)
    hin3 = hin.reshape(_NW, _NP // 128, 128).transpose(1, 0, 2)

    x1, so, si = _mm1(hout3, hin3, feats3, W1)
    p = _agg(x1, src2, dst2)
    x2 = _mm2(p[0], p[1], si, so, b1.reshape(1, _D2), W2)
    q = _agg(x2, src2, dst2)
    out = _out(q[0], q[1], si, b2.reshape(1, _D2))
    return out[:_N]


# speedup vs baseline: 2.8018x; 2.8018x over previous
import jax
import jax.numpy as jnp
from jax import lax
from jax.experimental import pallas as pl
from jax.experimental.pallas import tpu as pltpu
from jax.experimental.pallas import tpu_sc as plsc

_N = 10000
_NP = 10240
_E = 160000
_D1 = 256
_D2 = 128

_NC = 2
_NS = 16
_NW = _NC * _NS
_CH = 128
_EP = 163840
_NCH = _EP // _CH
_CPT = _NCH // _NW
_HPT = _CPT // 2
_RPT = _NP // _NS

_RB3 = 10
_RB = _RB3 * 128
_GRID = _NP // _RB

_mesh = plsc.VectorSubcoreMesh(core_axis_name="c", subcore_axis_name="s")


def _deg_body(src_hbm, dst_hbm, hout_hbm, hin_hbm, idxs_v, idxd_v, ho_v, hi_v):
    c = lax.axis_index("c")
    s = lax.axis_index("s")
    wid = c * _NS + s
    z16 = jnp.zeros((16,), jnp.float32)

    def zero(i, _):
        ho_v[pl.ds(i * 16, 16)] = z16
        hi_v[pl.ds(i * 16, 16)] = z16
        return 0

    lax.fori_loop(0, _NP // 16, zero, 0)

    pltpu.sync_copy(src_hbm.at[pl.ds(wid * _CPT, _CPT)], idxs_v)
    pltpu.sync_copy(dst_hbm.at[pl.ds(wid * _CPT, _CPT)], idxd_v)

    ones16 = jnp.ones((16,), jnp.float32)

    def body(t, _):
        for j in range(_CH // 16):
            plsc.addupdate_scatter(ho_v, [idxs_v[t, pl.ds(j * 16, 16)]], ones16)
            plsc.addupdate_scatter(hi_v, [idxd_v[t, pl.ds(j * 16, 16)]], ones16)
        return 0

    lax.fori_loop(0, _CPT, body, 0)
    pltpu.sync_copy(ho_v, hout_hbm.at[wid])
    pltpu.sync_copy(hi_v, hin_hbm.at[wid])


_deg = pl.kernel(
    _deg_body,
    out_type=(
        jax.ShapeDtypeStruct((_NW, _NP), jnp.float32),
        jax.ShapeDtypeStruct((_NW, _NP), jnp.float32),
    ),
    mesh=_mesh,
    compiler_params=pltpu.CompilerParams(needs_layout_passes=False),
    scratch_types=[
        pltpu.VMEM((_CPT, _CH), jnp.int32),
        pltpu.VMEM((_CPT, _CH), jnp.int32),
        pltpu.VMEM((_NP,), jnp.float32),
        pltpu.VMEM((_NP,), jnp.float32),
    ],
)


def _agg_body(x_hbm, src_hbm, dst_hbm, out_hbm,
              idxs_v, idxd_v, buf0, buf1, acc_sh, gs0, gs1, ss0, ss1):
    c = lax.axis_index("c")
    s = lax.axis_index("s")
    wid = c * _NS + s
    cb = wid * _CPT
    z16 = jnp.zeros((16,), jnp.float32)

    def zero(i, _):
        for j in range(_D2 // 16):
            buf0[i, pl.ds(j * 16, 16)] = z16
        return 0

    lax.fori_loop(0, _CH, zero, 0)

    pltpu.sync_copy(src_hbm.at[pl.ds(cb, _CPT)], idxs_v)
    pltpu.sync_copy(dst_hbm.at[pl.ds(cb, _CPT)], idxd_v)

    base = s * _RPT
    for k in range(_RPT // _CH):
        pltpu.sync_copy(buf0, acc_sh.at[pl.ds(base + k * _CH, _CH)])
    plsc.subcore_barrier()

    pltpu.async_copy(x_hbm.at[idxs_v.at[0]], buf0, gs0)

    def body(t, _):
        j0 = 2 * t
        j1 = 2 * t + 1
        pltpu.make_async_copy(x_hbm.at[idxs_v.at[j0]], buf0, gs0).wait()
        h_s0 = pltpu.async_copy(buf0, acc_sh.at[idxd_v.at[j0]], ss0, add=True)
        h_g1 = pltpu.async_copy(x_hbm.at[idxs_v.at[j1]], buf1, gs1)
        h_g1.wait()
        h_s1 = pltpu.async_copy(buf1, acc_sh.at[idxd_v.at[j1]], ss1, add=True)
        h_s0.wait()

        @pl.when(t + 1 < _HPT)
        def _():
            pltpu.async_copy(x_hbm.at[idxs_v.at[j0 + 2]], buf0, gs0)

        h_s1.wait()
        return 0

    lax.fori_loop(0, _HPT, body, 0)
    plsc.subcore_barrier()
    pltpu.sync_copy(acc_sh.at[pl.ds(base, _RPT)], out_hbm.at[c, pl.ds(base, _RPT)])


_agg = pl.kernel(
    _agg_body,
    out_type=jax.ShapeDtypeStruct((_NC, _NP, _D2), jnp.float32),
    mesh=_mesh,
    scratch_types=[
        pltpu.VMEM((_CPT, _CH), jnp.int32),
        pltpu.VMEM((_CPT, _CH), jnp.int32),
        pltpu.VMEM((_CH, _D2), jnp.float32),
        pltpu.VMEM((_CH, _D2), jnp.float32),
        pltpu.VMEM_SHARED((_NP, _D2), jnp.float32),
        pltpu.SemaphoreType.DMA,
        pltpu.SemaphoreType.DMA,
        pltpu.SemaphoreType.DMA,
        pltpu.SemaphoreType.DMA,
    ],
)


def _mm1_body(hout_ref, hin_ref, x_ref, w_ref, y_ref, so_ref, si_ref):
    do = jnp.sum(hout_ref[...], axis=1)
    di = jnp.sum(hin_ref[...], axis=1)
    so = lax.rsqrt(jnp.maximum(do, 1.0))
    si = lax.rsqrt(jnp.maximum(di, 1.0))
    so_ref[...] = so[None]
    si_ref[...] = si[None]
    x = x_ref[...] * so[:, :, None]
    y_ref[...] = jnp.dot(
        x.reshape(_RB, _D1), w_ref[...], preferred_element_type=jnp.float32
    )


def _mm1(hout3, hin3, feats3, w1):
    return pl.pallas_call(
        _mm1_body,
        grid=(_GRID,),
        in_specs=[
            pl.BlockSpec((_RB3, _NW, 128), lambda b: (b, 0, 0)),
            pl.BlockSpec((_RB3, _NW, 128), lambda b: (b, 0, 0)),
            pl.BlockSpec((_RB3, 128, _D1), lambda b: (b, 0, 0)),
            pl.BlockSpec((_D1, _D2), lambda b: (0, 0)),
        ],
        out_specs=[
            pl.BlockSpec((_RB, _D2), lambda b: (b, 0)),
            pl.BlockSpec((1, _RB3, 128), lambda b: (b, 0, 0)),
            pl.BlockSpec((1, _RB3, 128), lambda b: (b, 0, 0)),
        ],
        out_shape=[
            jax.ShapeDtypeStruct((_NP, _D2), jnp.float32),
            jax.ShapeDtypeStruct((_GRID, _RB3, 128), jnp.float32),
            jax.ShapeDtypeStruct((_GRID, _RB3, 128), jnp.float32),
        ],
    )(hout3, hin3, feats3, w1)


def _mm2_body(p0_ref, p1_ref, si_ref, so_ref, b1_ref, w_ref, y_ref):
    p = (p0_ref[...] + p1_ref[...]).reshape(_RB3, 128, _D2)
    si = si_ref[0]
    so = so_ref[0]
    b = b1_ref[...]
    h = jnp.maximum(p * si[:, :, None] + b[0][None, None, :], 0.0)
    h = h * so[:, :, None]
    y_ref[...] = jnp.dot(
        h.reshape(_RB, _D2), w_ref[...], preferred_element_type=jnp.float32
    )


def _mm2(p0, p1, si, so, b1_2d, w2):
    return pl.pallas_call(
        _mm2_body,
        grid=(_GRID,),
        in_specs=[
            pl.BlockSpec((_RB, _D2), lambda b: (b, 0)),
            pl.BlockSpec((_RB, _D2), lambda b: (b, 0)),
            pl.BlockSpec((1, _RB3, 128), lambda b: (b, 0, 0)),
            pl.BlockSpec((1, _RB3, 128), lambda b: (b, 0, 0)),
            pl.BlockSpec((1, _D2), lambda b: (0, 0)),
            pl.BlockSpec((_D2, _D2), lambda b: (0, 0)),
        ],
        out_specs=pl.BlockSpec((_RB, _D2), lambda b: (b, 0)),
        out_shape=jax.ShapeDtypeStruct((_NP, _D2), jnp.float32),
    )(p0, p1, si, so, b1_2d, w2)


def _out_body(q0_ref, q1_ref, si_ref, b2_ref, o_ref):
    q = (q0_ref[...] + q1_ref[...]).reshape(_RB3, 128, _D2)
    si = si_ref[0]
    b = b2_ref[...]
    o = jnp.maximum(q * si[:, :, None] + b[0][None, None, :], 0.0)
    o_ref[...] = o.reshape(_RB, _D2)


def _out(q0, q1, si, b2_2d):
    return pl.pallas_call(
        _out_body,
        grid=(_GRID,),
        in_specs=[
            pl.BlockSpec((_RB, _D2), lambda b: (b, 0)),
            pl.BlockSpec((_RB, _D2), lambda b: (b, 0)),
            pl.BlockSpec((1, _RB3, 128), lambda b: (b, 0, 0)),
            pl.BlockSpec((1, _D2), lambda b: (0, 0)),
        ],
        out_specs=pl.BlockSpec((_RB, _D2), lambda b: (b, 0)),
        out_shape=jax.ShapeDtypeStruct((_NP, _D2), jnp.float32),
    )(q0, q1, si, b2_2d)


def kernel(feats, edge_index, W1, b1, W2, b2):
    src = edge_index[0].astype(jnp.int32)
    dst = edge_index[1].astype(jnp.int32)
    pad = _N + (jnp.arange(_EP - _E, dtype=jnp.int32) % (_NP - _N))
    src2 = jnp.concatenate([src, pad]).reshape(_NCH, _CH)
    dst2 = jnp.concatenate([dst, pad]).reshape(_NCH, _CH)
    feats3 = jnp.pad(feats, ((0, _NP - _N), (0, 0))).reshape(_NP // 128, 128, _D1)

    hout, hin = _deg(src2, dst2)
    hout3 = hout.reshape(_NW, _NP // 128, 128).transpose(1, 0, 2)
    hin3 = hin.reshape(_NW, _NP // 128, 128).transpose(1, 0, 2)

    x1, so, si = _mm1(hout3, hin3, feats3, W1)
    p = _agg(x1, src2, dst2)
    x2 = _mm2(p[0], p[1], si, so, b1.reshape(1, _D2), W2)
    q = _agg(x2, src2, dst2)
    out = _out(q[0], q[1], si, b2.reshape(1, _D2))
    return out[:_N]


# speedup vs baseline: 2.8800x; 1.0279x over previous
import jax
import jax.numpy as jnp
from jax import lax
from jax.experimental import pallas as pl
from jax.experimental.pallas import tpu as pltpu
from jax.experimental.pallas import tpu_sc as plsc

_N = 10000
_NP = 10240
_E = 160000
_D1 = 256
_D2 = 128

_NC = 2
_NS = 16
_NW = _NC * _NS
_CH = 128
_EP = 163840
_NCH = _EP // _CH
_CPT = _NCH // _NW
_RPT = _NP // _NS

_NB = 2
_SPT = _CPT // _NB

_RB3 = 10
_RB = _RB3 * 128
_GRID = _NP // _RB

_mesh = plsc.VectorSubcoreMesh(core_axis_name="c", subcore_axis_name="s")


def _deg_body(src_hbm, dst_hbm, hout_hbm, hin_hbm, idxs_v, idxd_v, ho_v, hi_v):
    c = lax.axis_index("c")
    s = lax.axis_index("s")
    wid = c * _NS + s
    z16 = jnp.zeros((16,), jnp.float32)

    def zero(i, _):
        for j in range(128 // 16):
            ho_v[i, pl.ds(j * 16, 16)] = z16
            hi_v[i, pl.ds(j * 16, 16)] = z16
        return 0

    lax.fori_loop(0, _NP // 128, zero, 0)

    pltpu.sync_copy(src_hbm.at[pl.ds(wid * _CPT, _CPT)], idxs_v)
    pltpu.sync_copy(dst_hbm.at[pl.ds(wid * _CPT, _CPT)], idxd_v)

    ones16 = jnp.ones((16,), jnp.float32)

    def body(t, _):
        for j in range(_CH // 16):
            iv = idxs_v[t, pl.ds(j * 16, 16)]
            plsc.addupdate_scatter(ho_v, [iv >> 7, iv & 127], ones16)
            iv = idxd_v[t, pl.ds(j * 16, 16)]
            plsc.addupdate_scatter(hi_v, [iv >> 7, iv & 127], ones16)
        return 0

    lax.fori_loop(0, _CPT, body, 0)
    pltpu.sync_copy(ho_v, hout_hbm.at[:, wid])
    pltpu.sync_copy(hi_v, hin_hbm.at[:, wid])


_deg = pl.kernel(
    _deg_body,
    out_type=(
        jax.ShapeDtypeStruct((_NP // 128, _NW, 128), jnp.float32),
        jax.ShapeDtypeStruct((_NP // 128, _NW, 128), jnp.float32),
    ),
    mesh=_mesh,
    compiler_params=pltpu.CompilerParams(needs_layout_passes=False),
    scratch_types=[
        pltpu.VMEM((_CPT, _CH), jnp.int32),
        pltpu.VMEM((_CPT, _CH), jnp.int32),
        pltpu.VMEM((_NP // 128, 128), jnp.float32),
        pltpu.VMEM((_NP // 128, 128), jnp.float32),
    ],
)


def _agg_body(x_hbm, src_hbm, dst_hbm, out_hbm,
              idxs_v, idxd_v, bufs, acc_sh, gsems, ssems):
    c = lax.axis_index("c")
    s = lax.axis_index("s")
    wid = c * _NS + s
    cb = wid * _CPT
    z16 = jnp.zeros((16,), jnp.float32)
    buf0 = bufs[0]
    buf1 = bufs[1]

    pltpu.sync_copy(src_hbm.at[pl.ds(cb, _CPT)], idxs_v)
    pltpu.sync_copy(dst_hbm.at[pl.ds(cb, _CPT)], idxd_v)
    pltpu.async_copy(x_hbm.at[idxs_v.at[0]], buf0, gsems[0])

    def zero(i, _):
        for j in range(_D2 // 16):
            buf1[i, pl.ds(j * 16, 16)] = z16
        return 0

    lax.fori_loop(0, _CH, zero, 0)

    base = s * _RPT
    for k in range(_RPT // _CH):
        pltpu.sync_copy(buf1, acc_sh.at[pl.ds(base + k * _CH, _CH)])
    plsc.subcore_barrier()


    def body(t, _):
        j0 = 2 * t
        j1 = 2 * t + 1
        pltpu.make_async_copy(x_hbm.at[idxs_v.at[j0]], buf0, gsems[0]).wait()
        h_s0 = pltpu.async_copy(buf0, acc_sh.at[idxd_v.at[j0]], ssems[0], add=True)
        h_g1 = pltpu.async_copy(x_hbm.at[idxs_v.at[j1]], buf1, gsems[1])
        h_g1.wait()
        h_s1 = pltpu.async_copy(buf1, acc_sh.at[idxd_v.at[j1]], ssems[1], add=True)
        h_s0.wait()

        @pl.when(t + 1 < _SPT)
        def _():
            pltpu.async_copy(x_hbm.at[idxs_v.at[j0 + 2]], buf0, gsems[0])

        h_s1.wait()
        return 0

    lax.fori_loop(0, _SPT, body, 0)
    plsc.subcore_barrier()
    pltpu.sync_copy(acc_sh.at[pl.ds(base, _RPT)], out_hbm.at[c, pl.ds(base, _RPT)])


_agg = pl.kernel(
    _agg_body,
    out_type=jax.ShapeDtypeStruct((_NC, _NP, _D2), jnp.float32),
    mesh=_mesh,
    scratch_types=[
        pltpu.VMEM((_CPT, _CH), jnp.int32),
        pltpu.VMEM((_CPT, _CH), jnp.int32),
        [pltpu.VMEM((_CH, _D2), jnp.float32) for _ in range(_NB)],
        pltpu.VMEM_SHARED((_NP, _D2), jnp.float32),
        [pltpu.SemaphoreType.DMA for _ in range(_NB)],
        [pltpu.SemaphoreType.DMA for _ in range(_NB)],
    ],
)


def _mm1_body(hout_ref, hin_ref, x_ref, w_ref, y_ref, so_ref, si_ref):
    do = jnp.sum(hout_ref[...], axis=1)
    di = jnp.sum(hin_ref[...], axis=1)
    so = lax.rsqrt(jnp.maximum(do, 1.0))
    si = lax.rsqrt(jnp.maximum(di, 1.0))
    so_ref[...] = so[None]
    si_ref[...] = si[None]
    x = x_ref[...] * so[:, :, None]
    y_ref[...] = jnp.dot(
        x.reshape(_RB, _D1), w_ref[...], preferred_element_type=jnp.float32
    )


def _mm1(hout3, hin3, feats3, w1):
    return pl.pallas_call(
        _mm1_body,
        grid=(_GRID,),
        compiler_params=pltpu.CompilerParams(
            allow_input_fusion=[False, False, True, False]),
        in_specs=[
            pl.BlockSpec((_RB3, _NW, 128), lambda b: (b, 0, 0)),
            pl.BlockSpec((_RB3, _NW, 128), lambda b: (b, 0, 0)),
            pl.BlockSpec((_RB3, 128, _D1), lambda b: (b, 0, 0)),
            pl.BlockSpec((_D1, _D2), lambda b: (0, 0)),
        ],
        out_specs=[
            pl.BlockSpec((_RB, _D2), lambda b: (b, 0)),
            pl.BlockSpec((1, _RB3, 128), lambda b: (b, 0, 0)),
            pl.BlockSpec((1, _RB3, 128), lambda b: (b, 0, 0)),
        ],
        out_shape=[
            jax.ShapeDtypeStruct((_NP, _D2), jnp.float32),
            jax.ShapeDtypeStruct((_GRID, _RB3, 128), jnp.float32),
            jax.ShapeDtypeStruct((_GRID, _RB3, 128), jnp.float32),
        ],
    )(hout3, hin3, feats3, w1)


def _mm2_body(p0_ref, p1_ref, si_ref, so_ref, b1_ref, w_ref, y_ref):
    p = (p0_ref[...] + p1_ref[...]).reshape(_RB3, 128, _D2)
    si = si_ref[0]
    so = so_ref[0]
    b = b1_ref[...]
    h = jnp.maximum(p * si[:, :, None] + b[0][None, None, :], 0.0)
    h = h * so[:, :, None]
    y_ref[...] = jnp.dot(
        h.reshape(_RB, _D2), w_ref[...], preferred_element_type=jnp.float32
    )


def _mm2(p0, p1, si, so, b1_2d, w2):
    return pl.pallas_call(
        _mm2_body,
        grid=(_GRID,),
        in_specs=[
            pl.BlockSpec((_RB, _D2), lambda b: (b, 0)),
            pl.BlockSpec((_RB, _D2), lambda b: (b, 0)),
            pl.BlockSpec((1, _RB3, 128), lambda b: (b, 0, 0)),
            pl.BlockSpec((1, _RB3, 128), lambda b: (b, 0, 0)),
            pl.BlockSpec((1, _D2), lambda b: (0, 0)),
            pl.BlockSpec((_D2, _D2), lambda b: (0, 0)),
        ],
        out_specs=pl.BlockSpec((_RB, _D2), lambda b: (b, 0)),
        out_shape=jax.ShapeDtypeStruct((_NP, _D2), jnp.float32),
    )(p0, p1, si, so, b1_2d, w2)


def _out_body(q0_ref, q1_ref, si_ref, b2_ref, o_ref):
    q = (q0_ref[...] + q1_ref[...]).reshape(_RB3, 128, _D2)
    si = si_ref[0]
    b = b2_ref[...]
    o = jnp.maximum(q * si[:, :, None] + b[0][None, None, :], 0.0)
    o_ref[...] = o.reshape(_RB, _D2)


def _out(q0, q1, si, b2_2d):
    return pl.pallas_call(
        _out_body,
        grid=(_GRID,),
        in_specs=[
            pl.BlockSpec((_RB, _D2), lambda b: (b, 0)),
            pl.BlockSpec((_RB, _D2), lambda b: (b, 0)),
            pl.BlockSpec((1, _RB3, 128), lambda b: (b, 0, 0)),
            pl.BlockSpec((1, _D2), lambda b: (0, 0)),
        ],
        out_specs=pl.BlockSpec((_RB, _D2), lambda b: (b, 0)),
        out_shape=jax.ShapeDtypeStruct((_NP, _D2), jnp.float32),
    )(q0, q1, si, b2_2d)


def kernel(feats, edge_index, W1, b1, W2, b2):
    src = edge_index[0].astype(jnp.int32)
    dst = edge_index[1].astype(jnp.int32)
    pad = _N + (jnp.arange(_EP - _E, dtype=jnp.int32) % (_NP - _N))
    src2 = jnp.concatenate([src, pad]).reshape(_NCH, _CH)
    dst2 = jnp.concatenate([dst, pad]).reshape(_NCH, _CH)
    feats3 = jnp.pad(feats, ((0, _NP - _N), (0, 0))).reshape(_NP // 128, 128, _D1)

    hout3, hin3 = _deg(src2, dst2)

    x1, so, si = _mm1(hout3, hin3, feats3, W1)
    p = _agg(x1, src2, dst2)
    x2 = _mm2(p[0], p[1], si, so, b1.reshape(1, _D2), W2)
    q = _agg(x2, src2, dst2)
    out = _out(q[0], q[1], si, b2.reshape(1, _D2))
    return out[:_N]
